# Initial kernel scaffold; baseline (speedup 1.0000x reference)
import jax
import jax.numpy as jnp
from jax.experimental import pallas as pl


def kernel(input, target, embed):
    raise NotImplementedError("write your pallas kernel here")


# speedup vs baseline: 1.0859x; 1.0859x over previous
import functools

import jax
import jax.numpy as jnp
from jax import lax
from jax.experimental import pallas as pl
from jax.experimental.pallas import tpu as pltpu
from jax.experimental.pallas import tpu_sc as plsc

V = 1000
N = 64 * 256
NC, NS = 2, 16
NW = NC * NS
ROWS_PER_W = N // NW
CHUNK = 64
NCHUNK = ROWS_PER_W // CHUNK
GROUPS = CHUNK // 16


def _logz_body(e_ref, o_ref):
    x = e_ref[...]
    m = jnp.max(x, axis=1)
    s = jnp.sum(jnp.exp(x - m[:, None]), axis=1)
    o_ref[...] = m + jnp.log(s)


def _loss_body(p_ref, o_ref):
    o_ref[0, 0] = jnp.sum(p_ref[...]) * (1.0 / N)


def _sc_body(embed, idx, tgt, logz, out, partials,
             idx_v, tgt_v, logz_v, rows_v, acc_v, sem):
    wid = lax.axis_index("s") * NC + lax.axis_index("c")
    base = wid * ROWS_PER_W
    pltpu.sync_copy(idx.at[pl.ds(base, ROWS_PER_W)], idx_v)
    pltpu.sync_copy(tgt.at[pl.ds(base, ROWS_PER_W)], tgt_v)
    pltpu.sync_copy(logz, logz_v)

    acc = jnp.zeros((16,), jnp.float32)
    for c in range(NCHUNK):
        pltpu.async_copy(embed.at[idx_v.at[pl.ds(c * CHUNK, CHUNK)]],
                         rows_v, sem).wait()
        pltpu.sync_copy(rows_v, out.at[pl.ds(base + c * CHUNK, CHUNK)])
        for g in range(GROUPS):
            off = c * CHUNK + g * 16
            idx16 = idx_v[pl.ds(off, 16)]
            tgt16 = tgt_v[pl.ds(off, 16)]
            rowids = jnp.arange(16, dtype=jnp.int32) + g * 16
            picked = plsc.load_gather(rows_v, [rowids, tgt16])
            lz = plsc.load_gather(logz_v, [idx16])
            acc = acc + (lz - picked)
    acc_v[...] = acc
    pltpu.sync_copy(acc_v, partials.at[wid])


_sc_gather = functools.partial(
    pl.kernel,
    mesh=plsc.VectorSubcoreMesh(core_axis_name="c", subcore_axis_name="s"),
    compiler_params=pltpu.CompilerParams(
        use_tc_tiling_on_sc=False, needs_layout_passes=False),
    out_type=[
        jax.ShapeDtypeStruct((N, V), jnp.float32),
        jax.ShapeDtypeStruct((NW, 16), jnp.float32),
    ],
    scratch_types=[
        pltpu.VMEM((ROWS_PER_W,), jnp.int32),
        pltpu.VMEM((ROWS_PER_W,), jnp.int32),
        pltpu.VMEM((V,), jnp.float32),
        pltpu.VMEM((CHUNK, V), jnp.float32),
        pltpu.VMEM((16,), jnp.float32),
        pltpu.SemaphoreType.DMA,
    ],
)(_sc_body)


def kernel(input, target, embed):
    idx = input.reshape(-1).astype(jnp.int32)
    tgt = target.reshape(-1).astype(jnp.int32)
    logz = pl.pallas_call(
        _logz_body,
        out_shape=jax.ShapeDtypeStruct((V,), jnp.float32),
    )(embed)
    logit2, partials = _sc_gather(embed, idx, tgt, logz)
    loss2d = pl.pallas_call(
        _loss_body,
        out_shape=jax.ShapeDtypeStruct((1, 1), jnp.float32),
        out_specs=pl.BlockSpec(memory_space=pltpu.SMEM),
    )(partials)
    return (logit2, loss2d[0, 0])


# speedup vs baseline: 1.1013x; 1.0142x over previous
import functools

import jax
import jax.numpy as jnp
from jax import lax
from jax.experimental import pallas as pl
from jax.experimental.pallas import tpu as pltpu
from jax.experimental.pallas import tpu_sc as plsc

V = 1000
N = 64 * 256
NC, NS = 2, 16
NW = NC * NS
ROWS_PER_W = N // NW
CHUNK = 64
NCHUNK = ROWS_PER_W // CHUNK
GROUPS = CHUNK // 16


def _logz_body(e_ref, o_ref):
    x = e_ref[...]
    m = jnp.max(x, axis=1)
    s = jnp.sum(jnp.exp(x - m[:, None]), axis=1)
    o_ref[...] = m + jnp.log(s)


def _loss_body(p_ref, o_ref):
    o_ref[0, 0] = jnp.sum(p_ref[...]) * (1.0 / N)


def _sc_body(embed, idx, tgt, logz, out, partials,
             idx_v, tgt_v, logz_v, rows0_v, rows1_v, acc_v,
             sem_i0, sem_i1, sem_o0, sem_o1):
    wid = lax.axis_index("s") * NC + lax.axis_index("c")
    base = wid * ROWS_PER_W
    pltpu.sync_copy(idx.at[pl.ds(base, ROWS_PER_W)], idx_v)
    pltpu.sync_copy(tgt.at[pl.ds(base, ROWS_PER_W)], tgt_v)
    pltpu.sync_copy(logz, logz_v)

    bufs = [rows0_v, rows1_v]
    sems_i = [sem_i0, sem_i1]
    sems_o = [sem_o0, sem_o1]

    def gather(c):
        return pltpu.async_copy(
            embed.at[idx_v.at[pl.ds(c * CHUNK, CHUNK)]], bufs[c % 2],
            sems_i[c % 2])

    acc = jnp.zeros((16,), jnp.float32)
    copies_in = {0: gather(0)}
    copies_out = {}
    for c in range(NCHUNK):
        b = c % 2
        if c + 1 < NCHUNK:
            if c - 1 >= 0:
                copies_out.pop(c - 1).wait()
            copies_in[c + 1] = gather(c + 1)
        copies_in.pop(c).wait()
        for g in range(GROUPS):
            off = c * CHUNK + g * 16
            idx16 = idx_v[pl.ds(off, 16)]
            tgt16 = tgt_v[pl.ds(off, 16)]
            rowids = jnp.arange(16, dtype=jnp.int32) + g * 16
            picked = plsc.load_gather(bufs[b], [rowids, tgt16])
            lz = plsc.load_gather(logz_v, [idx16])
            acc = acc + (lz - picked)
        copies_out[c] = pltpu.async_copy(
            bufs[b], out.at[pl.ds(base + c * CHUNK, CHUNK)], sems_o[b])
    copies_out.pop(NCHUNK - 2).wait()
    copies_out.pop(NCHUNK - 1).wait()
    acc_v[...] = acc
    pltpu.sync_copy(acc_v, partials.at[wid])


_sc_gather = functools.partial(
    pl.kernel,
    mesh=plsc.VectorSubcoreMesh(core_axis_name="c", subcore_axis_name="s"),
    compiler_params=pltpu.CompilerParams(
        use_tc_tiling_on_sc=False, needs_layout_passes=False),
    out_type=[
        jax.ShapeDtypeStruct((N, V), jnp.float32),
        jax.ShapeDtypeStruct((NW, 16), jnp.float32),
    ],
    scratch_types=[
        pltpu.VMEM((ROWS_PER_W,), jnp.int32),
        pltpu.VMEM((ROWS_PER_W,), jnp.int32),
        pltpu.VMEM((V,), jnp.float32),
        pltpu.VMEM((CHUNK, V), jnp.float32),
        pltpu.VMEM((CHUNK, V), jnp.float32),
        pltpu.VMEM((16,), jnp.float32),
        pltpu.SemaphoreType.DMA,
        pltpu.SemaphoreType.DMA,
        pltpu.SemaphoreType.DMA,
        pltpu.SemaphoreType.DMA,
    ],
)(_sc_body)


def kernel(input, target, embed):
    idx = input.reshape(-1).astype(jnp.int32)
    tgt = target.reshape(-1).astype(jnp.int32)
    logz = pl.pallas_call(
        _logz_body,
        out_shape=jax.ShapeDtypeStruct((V,), jnp.float32),
    )(embed)
    logit2, partials = _sc_gather(embed, idx, tgt, logz)
    loss2d = pl.pallas_call(
        _loss_body,
        out_shape=jax.ShapeDtypeStruct((1, 1), jnp.float32),
        out_specs=pl.BlockSpec(memory_space=pltpu.SMEM),
    )(partials)
    return (logit2, loss2d[0, 0])
